# Initial kernel scaffold; baseline (speedup 1.0000x reference)
import jax
import jax.numpy as jnp
from jax.experimental import pallas as pl


def kernel(x, edge_index, W_enc, b_enc, W_convs, b_convs, gammas, betas):
    raise NotImplementedError("write your pallas kernel here")


# speedup vs baseline: 9.5782x; 9.5782x over previous
import functools

import jax
import jax.numpy as jnp
from jax import lax
from jax.experimental import pallas as pl
from jax.experimental.pallas import tpu as pltpu
from jax.experimental.pallas import tpu_sc as plsc

N = 10000
E = 320000
D = 128

NC = 2
NS = 16
NW = NC * NS
EW = E // NW
CH = 80
NCH = EW // CH
HW = 128

ROWA = 624
REM_BASE = ROWA * NS
REM = N - REM_BASE


def _slab(s):
    return pl.multiple_of(s * ROWA, 8)

def _mesh():
    return plsc.VectorSubcoreMesh(core_axis_name="c", subcore_axis_name="s",
                                  num_cores=NC, num_subcores=NS)


def _hist_body(dst_hbm, ones_hbm, zeros_hbm, out_hbm, dst_v, ones_v, acc_sh):
    c = lax.axis_index("c")
    s = lax.axis_index("s")
    wid = s * NC + c
    pltpu.sync_copy(ones_hbm, ones_v)
    b = _slab(s)
    pltpu.sync_copy(zeros_hbm.at[pl.ds(b, ROWA)], acc_sh.at[pl.ds(b, ROWA)])

    @pl.when(s == NS - 1)
    def _():
        pltpu.sync_copy(zeros_hbm.at[pl.ds(REM_BASE, REM)],
                        acc_sh.at[pl.ds(REM_BASE, REM)])

    plsc.subcore_barrier()

    def body(k, carry):
        base = pl.multiple_of(wid * EW + k * CH, 8)
        pltpu.sync_copy(dst_hbm.at[pl.ds(base, CH)], dst_v)
        pltpu.sync_copy(ones_v, acc_sh.at[dst_v], add=True)
        return carry

    lax.fori_loop(0, NCH, body, 0)
    plsc.subcore_barrier()
    pltpu.sync_copy(acc_sh.at[pl.ds(b, ROWA)], out_hbm.at[c, pl.ds(b, ROWA)])

    @pl.when(s == NS - 1)
    def _():
        pltpu.sync_copy(acc_sh.at[pl.ds(REM_BASE, REM)],
                        out_hbm.at[c, pl.ds(REM_BASE, REM)])


@functools.lru_cache(maxsize=None)
def _hist_kernel():
    return pl.kernel(
        _hist_body,
        out_type=jax.ShapeDtypeStruct((NC, N, HW), jnp.float32),
        mesh=_mesh(),
        scratch_types=[
            pltpu.VMEM((CH,), jnp.int32),
            pltpu.VMEM((CH, HW), jnp.float32),
            pltpu.VMEM_SHARED((N, HW), jnp.float32),
        ],
    )


def _spmm_body(g_hbm, src_hbm, dst_hbm, zeros_hbm, out_hbm, src_v, dst_v,
               rows_v, acc_sh, sem):
    c = lax.axis_index("c")
    s = lax.axis_index("s")
    wid = s * NC + c
    b = _slab(s)
    pltpu.sync_copy(zeros_hbm.at[pl.ds(b, ROWA)], acc_sh.at[pl.ds(b, ROWA)])

    @pl.when(s == NS - 1)
    def _():
        pltpu.sync_copy(zeros_hbm.at[pl.ds(REM_BASE, REM)],
                        acc_sh.at[pl.ds(REM_BASE, REM)])

    plsc.subcore_barrier()

    def body(k, carry):
        base = pl.multiple_of(wid * EW + k * CH, 8)
        pltpu.sync_copy(src_hbm.at[pl.ds(base, CH)], src_v)
        pltpu.sync_copy(dst_hbm.at[pl.ds(base, CH)], dst_v)
        pltpu.async_copy(g_hbm.at[src_v], rows_v, sem).wait()
        pltpu.sync_copy(rows_v, acc_sh.at[dst_v], add=True)
        return carry

    lax.fori_loop(0, NCH, body, 0)
    plsc.subcore_barrier()
    pltpu.sync_copy(acc_sh.at[pl.ds(b, ROWA)], out_hbm.at[c, pl.ds(b, ROWA)])

    @pl.when(s == NS - 1)
    def _():
        pltpu.sync_copy(acc_sh.at[pl.ds(REM_BASE, REM)],
                        out_hbm.at[c, pl.ds(REM_BASE, REM)])


@functools.lru_cache(maxsize=None)
def _spmm_kernel():
    return pl.kernel(
        _spmm_body,
        out_type=jax.ShapeDtypeStruct((NC, N, D), jnp.float32),
        mesh=_mesh(),
        scratch_types=[
            pltpu.VMEM((CH,), jnp.int32),
            pltpu.VMEM((CH,), jnp.int32),
            pltpu.VMEM((CH, D), jnp.float32),
            pltpu.VMEM_SHARED((N, D), jnp.float32),
            pltpu.SemaphoreType.DMA,
        ],
    )


R = 1000
G = N // R

_row = pl.BlockSpec((R, D), lambda i: (i, 0))
_col = pl.BlockSpec((R, 1), lambda i: (i, 0))
_mat = pl.BlockSpec((D, D), lambda i: (0, 0))
_vec = pl.BlockSpec((1, D), lambda i: (0, 0))
_f32 = jnp.float32


def _ln_relu(h, gamma, beta):
    mu = jnp.mean(h, axis=-1, keepdims=True)
    var = jnp.mean((h - mu) ** 2, axis=-1, keepdims=True)
    t = (h - mu) * lax.rsqrt(var + 1e-5) * gamma + beta
    return jnp.maximum(t, 0.0)


def _d0_body(x_ref, we_ref, be_ref, w0_ref, deg_ref, g_ref):
    h = jnp.dot(x_ref[...], we_ref[...], preferred_element_type=_f32) + be_ref[...]
    hw = jnp.dot(h, w0_ref[...], preferred_element_type=_f32)
    g_ref[...] = hw * lax.rsqrt(deg_ref[...])


_d0 = pl.pallas_call(
    _d0_body,
    grid=(G,),
    in_specs=[_row, _mat, _vec, _mat, _col],
    out_specs=_row,
    out_shape=jax.ShapeDtypeStruct((N, D), _f32),
)


def _dmid_body(with_res, h_ref, g_ref, p0_ref, p1_ref, b_ref, ga_ref, be_ref,
               w_ref, deg_ref, hn_ref, gn_ref):
    dis = lax.rsqrt(deg_ref[...])
    h = dis * (p0_ref[...] + p1_ref[...] + g_ref[...]) + b_ref[...]
    if with_res:
        h = h + h_ref[...]
    t = _ln_relu(h, ga_ref[...], be_ref[...])
    hn_ref[...] = h
    gn_ref[...] = dis * jnp.dot(t, w_ref[...], preferred_element_type=_f32)


def _make_dmid(with_res):
    return pl.pallas_call(
        functools.partial(_dmid_body, with_res),
        grid=(G,),
        in_specs=[_row, _row, _row, _row, _vec, _vec, _vec, _mat, _col],
        out_specs=[_row, _row],
        out_shape=[jax.ShapeDtypeStruct((N, D), _f32),
                   jax.ShapeDtypeStruct((N, D), _f32)],
    )


_dmid_nores = _make_dmid(False)
_dmid_res = _make_dmid(True)


def _dlast_body(h_ref, g_ref, p0_ref, p1_ref, b_ref, ga_ref, be_ref, deg_ref,
                out_ref):
    dis = lax.rsqrt(deg_ref[...])
    h = h_ref[...] + dis * (p0_ref[...] + p1_ref[...] + g_ref[...]) + b_ref[...]
    out_ref[...] = _ln_relu(h, ga_ref[...], be_ref[...])


_dlast = pl.pallas_call(
    _dlast_body,
    grid=(G,),
    in_specs=[_row, _row, _row, _row, _vec, _vec, _vec, _col],
    out_specs=_row,
    out_shape=jax.ShapeDtypeStruct((N, D), _f32),
)


def kernel(x, edge_index, W_enc, b_enc, W_convs, b_convs, gammas, betas):
    src = edge_index[0]
    dst = edge_index[1]
    zeros_nd = jnp.zeros((N, D), _f32)
    ones_ch = jnp.ones((CH, HW), _f32)

    hist = _hist_kernel()(dst, ones_ch, zeros_nd)
    deg = hist[0, :, :1] + hist[1, :, :1] + 1.0

    b2 = lambda v: v.reshape(1, D)

    g = _d0(x, W_enc, b2(b_enc), W_convs[0], deg)
    P = _spmm_kernel()(g, src, dst, zeros_nd)
    h = None
    for i in range(1, 4):
        dm = _dmid_nores if i == 1 else _dmid_res
        args = (g, P[0], P[1], b2(b_convs[i - 1]), b2(gammas[i]),
                b2(betas[i]), W_convs[i], deg)
        if i == 1:
            h, g = dm(g * 0.0, *args)
        else:
            h, g = dm(h, *args)
        P = _spmm_kernel()(g, src, dst, zeros_nd)
    return _dlast(h, g, P[0], P[1], b2(b_convs[3]), b2(gammas[0]),
                  b2(betas[0]), deg)


# speedup vs baseline: 20.6694x; 2.1580x over previous
import functools

import jax
import jax.numpy as jnp
from jax import lax
from jax.experimental import pallas as pl
from jax.experimental.pallas import tpu as pltpu
from jax.experimental.pallas import tpu_sc as plsc

N = 10000
E = 320000
D = 128

NC = 2
NS = 16
NW = NC * NS
EW = E // NW
CH = 80
NCH = EW // CH
HW = 128

ROWA = 624
REM_BASE = ROWA * NS
REM = N - REM_BASE


def _slab(s):
    return pl.multiple_of(s * ROWA, 8)

def _mesh():
    return plsc.VectorSubcoreMesh(core_axis_name="c", subcore_axis_name="s",
                                  num_cores=NC, num_subcores=NS)


def _hist_body(dst_hbm, ones_hbm, zeros_hbm, out_hbm, dst_all, ones_v, acc_sh):
    c = lax.axis_index("c")
    s = lax.axis_index("s")
    wid = s * NC + c
    pltpu.sync_copy(ones_hbm, ones_v)
    pltpu.sync_copy(dst_hbm.at[wid], dst_all)
    b = _slab(s)
    pltpu.sync_copy(zeros_hbm.at[pl.ds(b, ROWA)], acc_sh.at[pl.ds(b, ROWA)])

    @pl.when(s == NS - 1)
    def _():
        pltpu.sync_copy(zeros_hbm.at[pl.ds(REM_BASE, REM)],
                        acc_sh.at[pl.ds(REM_BASE, REM)])

    plsc.subcore_barrier()

    def body(k, carry):
        pltpu.sync_copy(ones_v, acc_sh.at[dst_all.at[k]], add=True)
        return carry

    lax.fori_loop(0, NCH, body, 0)
    plsc.subcore_barrier()
    pltpu.sync_copy(acc_sh.at[pl.ds(b, ROWA)], out_hbm.at[c, pl.ds(b, ROWA)])

    @pl.when(s == NS - 1)
    def _():
        pltpu.sync_copy(acc_sh.at[pl.ds(REM_BASE, REM)],
                        out_hbm.at[c, pl.ds(REM_BASE, REM)])


@functools.lru_cache(maxsize=None)
def _hist_kernel():
    return pl.kernel(
        _hist_body,
        out_type=jax.ShapeDtypeStruct((NC, N, HW), jnp.float32),
        mesh=_mesh(),
        scratch_types=[
            pltpu.VMEM((NCH, CH), jnp.int32),
            pltpu.VMEM((CH, HW), jnp.float32),
            pltpu.VMEM_SHARED((N, HW), jnp.float32),
        ],
    )


NBUF = 2
NG = NCH // NBUF


def _spmm_body(g_hbm, src_hbm, dst_hbm, zeros_hbm, out_hbm, src_all, dst_all,
               rows, acc_sh, *sems):
    c = lax.axis_index("c")
    s = lax.axis_index("s")
    wid = s * NC + c
    b = _slab(s)
    ebase = pl.multiple_of(wid * EW, 8)
    pltpu.sync_copy(src_hbm.at[pl.ds(ebase, EW)], src_all)
    pltpu.sync_copy(dst_hbm.at[wid], dst_all)
    pltpu.sync_copy(zeros_hbm.at[pl.ds(b, ROWA)], acc_sh.at[pl.ds(b, ROWA)])

    @pl.when(s == NS - 1)
    def _():
        pltpu.sync_copy(zeros_hbm.at[pl.ds(REM_BASE, REM)],
                        acc_sh.at[pl.ds(REM_BASE, REM)])

    plsc.subcore_barrier()

    def fire(buf, k):
        pltpu.async_copy(g_hbm.at[src_all.at[pl.ds(k * CH, CH)]], rows.at[buf],
                         sems[buf])

    def wait(buf):
        pltpu.make_async_copy(g_hbm.at[pl.ds(0, CH)], rows.at[buf],
                              sems[buf]).wait()

    def scatter(buf, k):
        pltpu.sync_copy(rows.at[buf], acc_sh.at[dst_all.at[k]], add=True)

    for buf in range(NBUF):
        fire(buf, buf)

    def body(gidx, carry):
        for buf in range(NBUF):
            k = gidx * NBUF + buf
            wait(buf)
            scatter(buf, k)

            @pl.when(k + NBUF < NCH)
            def _():
                fire(buf, k + NBUF)

        return carry

    lax.fori_loop(0, NG, body, 0)
    for k in range(NG * NBUF, NCH):
        buf = k % NBUF
        wait(buf)
        scatter(buf, k)

    plsc.subcore_barrier()
    pltpu.sync_copy(acc_sh.at[pl.ds(b, ROWA)], out_hbm.at[c, pl.ds(b, ROWA)])

    @pl.when(s == NS - 1)
    def _():
        pltpu.sync_copy(acc_sh.at[pl.ds(REM_BASE, REM)],
                        out_hbm.at[c, pl.ds(REM_BASE, REM)])


@functools.lru_cache(maxsize=None)
def _spmm_kernel():
    return pl.kernel(
        _spmm_body,
        out_type=jax.ShapeDtypeStruct((NC, N, D), jnp.float32),
        mesh=_mesh(),
        scratch_types=[
            pltpu.VMEM((EW,), jnp.int32),
            pltpu.VMEM((NCH, CH), jnp.int32),
            pltpu.VMEM((NBUF, CH, D), jnp.float32),
            pltpu.VMEM_SHARED((N, D), jnp.float32),
        ] + [pltpu.SemaphoreType.DMA] * NBUF,
    )


R = 1000
G = N // R

_row = pl.BlockSpec((R, D), lambda i: (i, 0))
_col = pl.BlockSpec((R, 1), lambda i: (i, 0))
_mat = pl.BlockSpec((D, D), lambda i: (0, 0))
_vec = pl.BlockSpec((1, D), lambda i: (0, 0))
_f32 = jnp.float32


def _ln_relu(h, gamma, beta):
    mu = jnp.mean(h, axis=-1, keepdims=True)
    var = jnp.mean((h - mu) ** 2, axis=-1, keepdims=True)
    t = (h - mu) * lax.rsqrt(var + 1e-5) * gamma + beta
    return jnp.maximum(t, 0.0)


def _d0_body(x_ref, we_ref, be_ref, w0_ref, deg_ref, g_ref):
    h = jnp.dot(x_ref[...], we_ref[...], preferred_element_type=_f32) + be_ref[...]
    hw = jnp.dot(h, w0_ref[...], preferred_element_type=_f32)
    g_ref[...] = hw * lax.rsqrt(deg_ref[...])


_d0 = pl.pallas_call(
    _d0_body,
    grid=(G,),
    in_specs=[_row, _mat, _vec, _mat, _col],
    out_specs=_row,
    out_shape=jax.ShapeDtypeStruct((N, D), _f32),
)


def _dmid_body(with_res, h_ref, g_ref, p0_ref, p1_ref, b_ref, ga_ref, be_ref,
               w_ref, deg_ref, hn_ref, gn_ref):
    dis = lax.rsqrt(deg_ref[...])
    h = dis * (p0_ref[...] + p1_ref[...] + g_ref[...]) + b_ref[...]
    if with_res:
        h = h + h_ref[...]
    t = _ln_relu(h, ga_ref[...], be_ref[...])
    hn_ref[...] = h
    gn_ref[...] = dis * jnp.dot(t, w_ref[...], preferred_element_type=_f32)


def _make_dmid(with_res):
    return pl.pallas_call(
        functools.partial(_dmid_body, with_res),
        grid=(G,),
        in_specs=[_row, _row, _row, _row, _vec, _vec, _vec, _mat, _col],
        out_specs=[_row, _row],
        out_shape=[jax.ShapeDtypeStruct((N, D), _f32),
                   jax.ShapeDtypeStruct((N, D), _f32)],
    )


_dmid_nores = _make_dmid(False)
_dmid_res = _make_dmid(True)


def _dlast_body(h_ref, g_ref, p0_ref, p1_ref, b_ref, ga_ref, be_ref, deg_ref,
                out_ref):
    dis = lax.rsqrt(deg_ref[...])
    h = h_ref[...] + dis * (p0_ref[...] + p1_ref[...] + g_ref[...]) + b_ref[...]
    out_ref[...] = _ln_relu(h, ga_ref[...], be_ref[...])


_dlast = pl.pallas_call(
    _dlast_body,
    grid=(G,),
    in_specs=[_row, _row, _row, _row, _vec, _vec, _vec, _col],
    out_specs=_row,
    out_shape=jax.ShapeDtypeStruct((N, D), _f32),
)


def kernel(x, edge_index, W_enc, b_enc, W_convs, b_convs, gammas, betas):
    src = edge_index[0]
    dst = edge_index[1].reshape(NW, NCH, CH)
    zeros_nd = jnp.zeros((N, D), _f32)
    ones_ch = jnp.ones((CH, HW), _f32)

    hist = _hist_kernel()(dst, ones_ch, zeros_nd)
    deg = hist[0, :, :1] + hist[1, :, :1] + 1.0

    b2 = lambda v: v.reshape(1, D)

    g = _d0(x, W_enc, b2(b_enc), W_convs[0], deg)
    P = _spmm_kernel()(g, src, dst, zeros_nd)
    h = None
    for i in range(1, 4):
        dm = _dmid_nores if i == 1 else _dmid_res
        args = (g, P[0], P[1], b2(b_convs[i - 1]), b2(gammas[i]),
                b2(betas[i]), W_convs[i], deg)
        if i == 1:
            h, g = dm(g * 0.0, *args)
        else:
            h, g = dm(h, *args)
        P = _spmm_kernel()(g, src, dst, zeros_nd)
    return _dlast(h, g, P[0], P[1], b2(b_convs[3]), b2(gammas[0]),
                  b2(betas[0]), deg)
